# baseline (speedup 1.0000x reference)
import functools

import jax
import jax.numpy as jnp
from jax.experimental import pallas as pl
from jax.experimental.pallas import tpu as pltpu

F32 = jnp.float32
EPS = 1e-16


def _rep_mat(a):
    H, C = a.shape
    HC = H * C
    jj = jnp.arange(HC)[:, None]
    ll = jnp.arange(HC)[None, :]
    return jnp.where(jj // C == ll // C, a[ll // C, jj % C], 0.0).astype(F32)


def _dense0_kernel(x_ref, w_ref, as_ref, ad_ref, xs_ref, xd_ref):
    xp = jnp.dot(x_ref[...], w_ref[...], preferred_element_type=F32)
    xs_ref[:, : xp.shape[1]] = xp
    xs_ref[:, xp.shape[1]:] = jnp.dot(xp, as_ref[...], preferred_element_type=F32)
    xd_ref[...] = jnp.dot(xp, ad_ref[...], preferred_element_type=F32)


def _dense_mid_kernel(acc_ref, b_ref, w_ref, as_ref, adr_ref, xs_ref, xd_ref):
    HC = acc_ref.shape[1] // 2
    h = jnp.maximum(acc_ref[:, :HC] / (acc_ref[:, HC:] + EPS) + b_ref[...], 0.0)
    xp = jnp.dot(h, w_ref[...], preferred_element_type=F32)
    xs_ref[:, : xp.shape[1]] = xp
    xs_ref[:, xp.shape[1]:] = jnp.dot(xp, as_ref[...], preferred_element_type=F32)
    xd_ref[...] = jnp.dot(xp, adr_ref[...], preferred_element_type=F32)


def _make_edge_kernel(HC, CH, n_valid, lo, hi, Nh):
    def ek(src_ref, dst_ref, xs_ref, xd_ref, acc_ref):
        g = pl.program_id(0)

        @pl.when(g == 0)
        def _init():
            acc_ref[...] = jnp.zeros_like(acc_ref)

        base = g * CH
        cmin = dst_ref[0, 0, 0]
        cmax = dst_ref[0, 0, CH - 1]

        @pl.when((cmax >= lo) & (cmin < hi))
        def _run():
            def body(i, carry):
                s = src_ref[0, 0, i]
                d = dst_ref[0, 0, i]
                dl = jnp.clip(d - lo, 0, Nh - 1)
                rs = xs_ref[pl.ds(s, 1), :]
                rd = xd_ref[pl.ds(dl, 1), :]
                al = rs[:, HC:] + rd
                al = jnp.where(al > 0, al, 0.2 * al)
                ea = jnp.exp(al)
                valid = (d >= lo) & (d < hi) & (base + i < n_valid)
                ea = jnp.where(valid, ea, 0.0)
                acc_ref[pl.ds(dl, 1), pl.ds(0, HC)] = (
                    acc_ref[pl.ds(dl, 1), pl.ds(0, HC)] + ea * rs[:, :HC])
                acc_ref[pl.ds(dl, 1), pl.ds(HC, HC)] = (
                    acc_ref[pl.ds(dl, 1), pl.ds(HC, HC)] + ea)
                return carry

            jax.lax.fori_loop(0, CH, body, 0)

    return ek


def _edge_pass(srcr, dstr, XS, XD, HC, CH, n_valid):
    N = XD.shape[0]
    G = srcr.shape[0]
    halves = []
    Nh = N // 2
    for lo in (0, Nh):
        ek = _make_edge_kernel(HC, CH, n_valid, lo, lo + Nh, Nh)
        acc = pl.pallas_call(
            ek,
            grid=(G,),
            in_specs=[
                pl.BlockSpec((1, 1, CH), lambda g: (g, 0, 0), memory_space=pltpu.SMEM),
                pl.BlockSpec((1, 1, CH), lambda g: (g, 0, 0), memory_space=pltpu.SMEM),
                pl.BlockSpec((N, 2 * HC), lambda g: (0, 0)),
                pl.BlockSpec((Nh, HC), lambda g: (0, 0)),
            ],
            out_specs=pl.BlockSpec((Nh, 2 * HC), lambda g: (0, 0)),
            out_shape=jax.ShapeDtypeStruct((Nh, 2 * HC), F32),
            compiler_params=pltpu.CompilerParams(
                vmem_limit_bytes=100 * 1024 * 1024,
            ),
        )(srcr, dstr, XS, jax.lax.slice_in_dim(XD, lo, lo + Nh))
        halves.append(acc)
    return jnp.concatenate(halves, axis=0)


def _heads_kernel(acc_ref, b_ref, x_ref,
                  pw1_ref, pb1_ref, pw2_ref, pb2_ref, pw3_ref, pb3_ref,
                  cw1_ref, cb1_ref, cw2_ref, cb2_ref,
                  h_ref, pc_ref, pp_ref, mg_ref, cc_ref, op_ref):
    HC = acc_ref.shape[1] // 2
    h = acc_ref[:, :HC] / (acc_ref[:, HC:] + EPS) + b_ref[...]
    h_ref[...] = h
    p = jnp.maximum(jnp.dot(h, pw1_ref[...], preferred_element_type=F32) + pb1_ref[...], 0.0)
    p = jnp.maximum(jnp.dot(p, pw2_ref[...], preferred_element_type=F32) + pb2_ref[...], 0.0)
    p = jnp.dot(p, pw3_ref[...], preferred_element_type=F32) + pb3_ref[...]
    pc_ref[...] = p
    pp_ref[...] = p + x_ref[...]
    mags = jnp.sqrt(jnp.sum(p * p, axis=1, keepdims=True))
    mg8 = jnp.broadcast_to(mags, mg_ref.shape)
    mg_ref[...] = mg8
    c = jnp.maximum(jnp.dot(h, cw1_ref[...], preferred_element_type=F32) + cb1_ref[...], 0.0)
    c = jax.nn.sigmoid(jnp.dot(c, cw2_ref[...], preferred_element_type=F32) + cb2_ref[...])
    cc_ref[...] = c
    op_ref[...] = (mg8 >= 2.5).astype(jnp.int32)


def _full(shape):
    return pl.BlockSpec(shape, lambda i: tuple(0 for _ in shape))


@functools.partial(jax.jit, static_argnames=())
def kernel(x, edge_index, W0, as0, ad0, b0, W1, as1, ad1, b1, W2, as2, ad2, b2,
           pcW1, pcb1, pcW2, pcb2, pcW3, pcb3, ccW1, ccb1, ccW2, ccb2):
    N = x.shape[0]
    E = edge_index.shape[1]
    Bn = 2000 if N % 2000 == 0 else N
    CH = 1024
    n_valid = E + N

    ei = edge_index.astype(jnp.int32)
    loop = jnp.arange(N, dtype=jnp.int32)
    src = jnp.concatenate([ei[0], loop])
    dst = jnp.concatenate([ei[1], loop])
    order = jnp.argsort(dst)
    src = src[order]
    dst = dst[order]
    G = (n_valid + CH - 1) // CH
    pad = G * CH - n_valid
    srcr = jnp.pad(src, (0, pad)).reshape(G, 1, CH)
    dstr = jnp.pad(dst, (0, pad), constant_values=N - 1).reshape(G, 1, CH)

    xpad = jnp.pad(x.astype(F32), ((0, 0), (0, 1)))
    W0p = jnp.pad(W0.astype(F32), ((0, 1), (0, 0)))
    As0, Ad0 = _rep_mat(as0[0]), _rep_mat(ad0[0])
    As1, Ad1 = _rep_mat(as1[0]), _rep_mat(ad1[0])
    As2, Ad2 = _rep_mat(as2[0]), _rep_mat(ad2[0])
    b0r, b1r, b2r = b0.reshape(1, 64), b1.reshape(1, 64), b2.reshape(1, 32)

    grid_n = N // Bn

    XS0, XD0 = pl.pallas_call(
        _dense0_kernel,
        grid=(grid_n,),
        in_specs=[
            pl.BlockSpec((Bn, 8), lambda i: (i, 0)),
            _full((8, 64)), _full((64, 64)), _full((64, 64)),
        ],
        out_specs=[pl.BlockSpec((Bn, 128), lambda i: (i, 0)),
                   pl.BlockSpec((Bn, 64), lambda i: (i, 0))],
        out_shape=[jax.ShapeDtypeStruct((N, 128), F32),
                   jax.ShapeDtypeStruct((N, 64), F32)],
    )(xpad, W0p, As0, Ad0)

    acc0 = _edge_pass(srcr, dstr, XS0, XD0, 64, CH, n_valid)

    XS1, XD1 = pl.pallas_call(
        _dense_mid_kernel,
        grid=(grid_n,),
        in_specs=[
            pl.BlockSpec((Bn, 128), lambda i: (i, 0)),
            _full((1, 64)), _full((64, 64)), _full((64, 64)), _full((64, 64)),
        ],
        out_specs=[pl.BlockSpec((Bn, 128), lambda i: (i, 0)),
                   pl.BlockSpec((Bn, 64), lambda i: (i, 0))],
        out_shape=[jax.ShapeDtypeStruct((N, 128), F32),
                   jax.ShapeDtypeStruct((N, 64), F32)],
    )(acc0, b0r, W1.astype(F32), As1, Ad1)

    acc1 = _edge_pass(srcr, dstr, XS1, XD1, 64, CH, n_valid)

    XS2, XD2 = pl.pallas_call(
        _dense_mid_kernel,
        grid=(grid_n,),
        in_specs=[
            pl.BlockSpec((Bn, 128), lambda i: (i, 0)),
            _full((1, 64)), _full((64, 32)), _full((32, 32)), _full((32, 32)),
        ],
        out_specs=[pl.BlockSpec((Bn, 64), lambda i: (i, 0)),
                   pl.BlockSpec((Bn, 32), lambda i: (i, 0))],
        out_shape=[jax.ShapeDtypeStruct((N, 64), F32),
                   jax.ShapeDtypeStruct((N, 32), F32)],
    )(acc1, b1r, W2.astype(F32), As2, Ad2)

    acc2 = _edge_pass(srcr, dstr, XS2, XD2, 32, CH, n_valid)

    pcW3p = jnp.pad(pcW3.astype(F32), ((0, 0), (0, 5)))
    pcb3p = jnp.pad(pcb3.astype(F32), (0, 5)).reshape(1, 8)
    ccW2p = jnp.pad(ccW2.astype(F32), ((0, 0), (0, 7)))
    ccb2p = jnp.pad(ccb2.astype(F32), (0, 7)).reshape(1, 8)
    x3p = jnp.pad(x[:, :3].astype(F32), ((0, 0), (0, 5)))

    houts = pl.pallas_call(
        _heads_kernel,
        grid=(grid_n,),
        in_specs=[
            pl.BlockSpec((Bn, 64), lambda i: (i, 0)),
            _full((1, 32)),
            pl.BlockSpec((Bn, 8), lambda i: (i, 0)),
            _full((32, 32)), _full((1, 32)),
            _full((32, 16)), _full((1, 16)),
            _full((16, 8)), _full((1, 8)),
            _full((32, 8)), _full((1, 8)),
            _full((8, 8)), _full((1, 8)),
        ],
        out_specs=[pl.BlockSpec((Bn, w), lambda i: (i, 0)) for w in (32, 8, 8, 8, 8, 8)],
        out_shape=[
            jax.ShapeDtypeStruct((N, 32), F32),
            jax.ShapeDtypeStruct((N, 8), F32),
            jax.ShapeDtypeStruct((N, 8), F32),
            jax.ShapeDtypeStruct((N, 8), F32),
            jax.ShapeDtypeStruct((N, 8), F32),
            jax.ShapeDtypeStruct((N, 8), jnp.int32),
        ],
    )(acc2, b2r, x3p,
      pcW1.astype(F32), pcb1.reshape(1, 32), pcW2.astype(F32), pcb2.reshape(1, 16),
      pcW3p, pcb3p, ccW1.astype(F32), ccb1.reshape(1, 8), ccW2p, ccb2p)

    h_out, pc8, pp8, mg8, cc8, op8 = houts
    pc = pc8[:, :3]
    mags = mg8[:, 0]
    cc = cc8[:, :1]
    pred_pos = pp8[:, :3]
    node_ops = op8[:, 0]
    return (pc, mags, cc, pred_pos, node_ops, h_out)


# speedup vs baseline: 1.0411x; 1.0411x over previous
import functools

import jax
import jax.numpy as jnp
from jax.experimental import pallas as pl
from jax.experimental.pallas import tpu as pltpu

F32 = jnp.float32
EPS = 1e-16


def _rep_mat(a):
    H, C = a.shape
    HC = H * C
    jj = jnp.arange(HC)[:, None]
    ll = jnp.arange(HC)[None, :]
    return jnp.where(jj // C == ll // C, a[ll // C, jj % C], 0.0).astype(F32)


def _dense0_kernel(x_ref, w_ref, as_ref, ad_ref, xs_ref, xd_ref):
    xp = jnp.dot(x_ref[...], w_ref[...], preferred_element_type=F32)
    xs_ref[:, : xp.shape[1]] = xp
    xs_ref[:, xp.shape[1]:] = jnp.dot(xp, as_ref[...], preferred_element_type=F32)
    xd_ref[...] = jnp.dot(xp, ad_ref[...], preferred_element_type=F32)


def _dense_mid_kernel(acc_ref, b_ref, w_ref, as_ref, adr_ref, xs_ref, xd_ref):
    HC = acc_ref.shape[1] // 2
    h = jnp.maximum(acc_ref[:, :HC] / (acc_ref[:, HC:] + EPS) + b_ref[...], 0.0)
    xp = jnp.dot(h, w_ref[...], preferred_element_type=F32)
    xs_ref[:, : xp.shape[1]] = xp
    xs_ref[:, xp.shape[1]:] = jnp.dot(xp, as_ref[...], preferred_element_type=F32)
    xd_ref[...] = jnp.dot(xp, adr_ref[...], preferred_element_type=F32)


def _make_edge_kernel(HC, CH, n_valid, lo, hi, Nh):
    def ek(src_ref, dst_ref, xs_ref, xd_ref, acc_ref):
        g = pl.program_id(0)

        @pl.when(g == 0)
        def _init():
            acc_ref[...] = jnp.zeros_like(acc_ref)

        base = g * CH
        cmin = dst_ref[0, 0, 0]
        cmax = dst_ref[0, 0, CH - 1]

        @pl.when((cmax >= lo) & (cmin < hi))
        def _run():
            def body(i, carry):
                s = src_ref[0, 0, i]
                d = dst_ref[0, 0, i]
                dl = jnp.clip(d - lo, 0, Nh - 1)
                rs = xs_ref[pl.ds(s, 1), :]
                rd = xd_ref[pl.ds(dl, 1), :]
                al = rs[:, HC:] + rd
                al = jnp.where(al > 0, al, 0.2 * al)
                ea = jnp.exp(al)
                valid = (d >= lo) & (d < hi) & (base + i < n_valid)
                ea = jnp.where(valid, ea, 0.0)
                upd = jnp.concatenate([ea * rs[:, :HC], ea], axis=1)
                acc_ref[pl.ds(dl, 1), :] = acc_ref[pl.ds(dl, 1), :] + upd
                return carry

            jax.lax.fori_loop(0, CH, body, 0)

    return ek


def _edge_pass(srcr, dstr, XS, XD, HC, CH, n_valid):
    N = XD.shape[0]
    G = srcr.shape[0]
    halves = []
    Nh = N // 2
    for lo in (0, Nh):
        ek = _make_edge_kernel(HC, CH, n_valid, lo, lo + Nh, Nh)
        acc = pl.pallas_call(
            ek,
            grid=(G,),
            in_specs=[
                pl.BlockSpec((1, 1, CH), lambda g: (g, 0, 0), memory_space=pltpu.SMEM),
                pl.BlockSpec((1, 1, CH), lambda g: (g, 0, 0), memory_space=pltpu.SMEM),
                pl.BlockSpec((N, 2 * HC), lambda g: (0, 0)),
                pl.BlockSpec((Nh, HC), lambda g: (0, 0)),
            ],
            out_specs=pl.BlockSpec((Nh, 2 * HC), lambda g: (0, 0)),
            out_shape=jax.ShapeDtypeStruct((Nh, 2 * HC), F32),
            compiler_params=pltpu.CompilerParams(
                vmem_limit_bytes=100 * 1024 * 1024,
            ),
        )(srcr, dstr, XS, jax.lax.slice_in_dim(XD, lo, lo + Nh))
        halves.append(acc)
    return jnp.concatenate(halves, axis=0)


def _heads_kernel(acc_ref, b_ref, x_ref,
                  pw1_ref, pb1_ref, pw2_ref, pb2_ref, pw3_ref, pb3_ref,
                  cw1_ref, cb1_ref, cw2_ref, cb2_ref,
                  h_ref, pc_ref, pp_ref, mg_ref, cc_ref, op_ref):
    HC = acc_ref.shape[1] // 2
    h = acc_ref[:, :HC] / (acc_ref[:, HC:] + EPS) + b_ref[...]
    h_ref[...] = h
    p = jnp.maximum(jnp.dot(h, pw1_ref[...], preferred_element_type=F32) + pb1_ref[...], 0.0)
    p = jnp.maximum(jnp.dot(p, pw2_ref[...], preferred_element_type=F32) + pb2_ref[...], 0.0)
    p = jnp.dot(p, pw3_ref[...], preferred_element_type=F32) + pb3_ref[...]
    pc_ref[...] = p
    pp_ref[...] = p + x_ref[...]
    mags = jnp.sqrt(jnp.sum(p * p, axis=1, keepdims=True))
    mg8 = jnp.broadcast_to(mags, mg_ref.shape)
    mg_ref[...] = mg8
    c = jnp.maximum(jnp.dot(h, cw1_ref[...], preferred_element_type=F32) + cb1_ref[...], 0.0)
    c = jax.nn.sigmoid(jnp.dot(c, cw2_ref[...], preferred_element_type=F32) + cb2_ref[...])
    cc_ref[...] = c
    op_ref[...] = (mg8 >= 2.5).astype(jnp.int32)


def _full(shape):
    return pl.BlockSpec(shape, lambda i: tuple(0 for _ in shape))


@functools.partial(jax.jit, static_argnames=())
def kernel(x, edge_index, W0, as0, ad0, b0, W1, as1, ad1, b1, W2, as2, ad2, b2,
           pcW1, pcb1, pcW2, pcb2, pcW3, pcb3, ccW1, ccb1, ccW2, ccb2):
    N = x.shape[0]
    E = edge_index.shape[1]
    Bn = 2000 if N % 2000 == 0 else N
    CH = 1024
    n_valid = E + N

    ei = edge_index.astype(jnp.int32)
    loop = jnp.arange(N, dtype=jnp.int32)
    src = jnp.concatenate([ei[0], loop])
    dst = jnp.concatenate([ei[1], loop])
    order = jnp.argsort(dst)
    src = src[order]
    dst = dst[order]
    G = (n_valid + CH - 1) // CH
    pad = G * CH - n_valid
    srcr = jnp.pad(src, (0, pad)).reshape(G, 1, CH)
    dstr = jnp.pad(dst, (0, pad), constant_values=N - 1).reshape(G, 1, CH)

    xpad = jnp.pad(x.astype(F32), ((0, 0), (0, 1)))
    W0p = jnp.pad(W0.astype(F32), ((0, 1), (0, 0)))
    As0, Ad0 = _rep_mat(as0[0]), _rep_mat(ad0[0])
    As1, Ad1 = _rep_mat(as1[0]), _rep_mat(ad1[0])
    As2, Ad2 = _rep_mat(as2[0]), _rep_mat(ad2[0])
    b0r, b1r, b2r = b0.reshape(1, 64), b1.reshape(1, 64), b2.reshape(1, 32)

    grid_n = N // Bn

    XS0, XD0 = pl.pallas_call(
        _dense0_kernel,
        grid=(grid_n,),
        in_specs=[
            pl.BlockSpec((Bn, 8), lambda i: (i, 0)),
            _full((8, 64)), _full((64, 64)), _full((64, 64)),
        ],
        out_specs=[pl.BlockSpec((Bn, 128), lambda i: (i, 0)),
                   pl.BlockSpec((Bn, 64), lambda i: (i, 0))],
        out_shape=[jax.ShapeDtypeStruct((N, 128), F32),
                   jax.ShapeDtypeStruct((N, 64), F32)],
    )(xpad, W0p, As0, Ad0)

    acc0 = _edge_pass(srcr, dstr, XS0, XD0, 64, CH, n_valid)

    XS1, XD1 = pl.pallas_call(
        _dense_mid_kernel,
        grid=(grid_n,),
        in_specs=[
            pl.BlockSpec((Bn, 128), lambda i: (i, 0)),
            _full((1, 64)), _full((64, 64)), _full((64, 64)), _full((64, 64)),
        ],
        out_specs=[pl.BlockSpec((Bn, 128), lambda i: (i, 0)),
                   pl.BlockSpec((Bn, 64), lambda i: (i, 0))],
        out_shape=[jax.ShapeDtypeStruct((N, 128), F32),
                   jax.ShapeDtypeStruct((N, 64), F32)],
    )(acc0, b0r, W1.astype(F32), As1, Ad1)

    acc1 = _edge_pass(srcr, dstr, XS1, XD1, 64, CH, n_valid)

    XS2, XD2 = pl.pallas_call(
        _dense_mid_kernel,
        grid=(grid_n,),
        in_specs=[
            pl.BlockSpec((Bn, 128), lambda i: (i, 0)),
            _full((1, 64)), _full((64, 32)), _full((32, 32)), _full((32, 32)),
        ],
        out_specs=[pl.BlockSpec((Bn, 64), lambda i: (i, 0)),
                   pl.BlockSpec((Bn, 32), lambda i: (i, 0))],
        out_shape=[jax.ShapeDtypeStruct((N, 64), F32),
                   jax.ShapeDtypeStruct((N, 32), F32)],
    )(acc1, b1r, W2.astype(F32), As2, Ad2)

    acc2 = _edge_pass(srcr, dstr, XS2, XD2, 32, CH, n_valid)

    pcW3p = jnp.pad(pcW3.astype(F32), ((0, 0), (0, 5)))
    pcb3p = jnp.pad(pcb3.astype(F32), (0, 5)).reshape(1, 8)
    ccW2p = jnp.pad(ccW2.astype(F32), ((0, 0), (0, 7)))
    ccb2p = jnp.pad(ccb2.astype(F32), (0, 7)).reshape(1, 8)
    x3p = jnp.pad(x[:, :3].astype(F32), ((0, 0), (0, 5)))

    houts = pl.pallas_call(
        _heads_kernel,
        grid=(grid_n,),
        in_specs=[
            pl.BlockSpec((Bn, 64), lambda i: (i, 0)),
            _full((1, 32)),
            pl.BlockSpec((Bn, 8), lambda i: (i, 0)),
            _full((32, 32)), _full((1, 32)),
            _full((32, 16)), _full((1, 16)),
            _full((16, 8)), _full((1, 8)),
            _full((32, 8)), _full((1, 8)),
            _full((8, 8)), _full((1, 8)),
        ],
        out_specs=[pl.BlockSpec((Bn, w), lambda i: (i, 0)) for w in (32, 8, 8, 8, 8, 8)],
        out_shape=[
            jax.ShapeDtypeStruct((N, 32), F32),
            jax.ShapeDtypeStruct((N, 8), F32),
            jax.ShapeDtypeStruct((N, 8), F32),
            jax.ShapeDtypeStruct((N, 8), F32),
            jax.ShapeDtypeStruct((N, 8), F32),
            jax.ShapeDtypeStruct((N, 8), jnp.int32),
        ],
    )(acc2, b2r, x3p,
      pcW1.astype(F32), pcb1.reshape(1, 32), pcW2.astype(F32), pcb2.reshape(1, 16),
      pcW3p, pcb3p, ccW1.astype(F32), ccb1.reshape(1, 8), ccW2p, ccb2p)

    h_out, pc8, pp8, mg8, cc8, op8 = houts
    pc = pc8[:, :3]
    mags = mg8[:, 0]
    cc = cc8[:, :1]
    pred_pos = pp8[:, :3]
    node_ops = op8[:, 0]
    return (pc, mags, cc, pred_pos, node_ops, h_out)


# speedup vs baseline: 2.0691x; 1.9875x over previous
import functools

import jax
import jax.numpy as jnp
from jax.experimental import pallas as pl
from jax.experimental.pallas import tpu as pltpu

F32 = jnp.float32
EPS = 1e-16


def _rep_mat(a):
    H, C = a.shape
    HC = H * C
    jj = jnp.arange(HC)[:, None]
    ll = jnp.arange(HC)[None, :]
    return jnp.where(jj // C == ll // C, a[ll // C, jj % C], 0.0).astype(F32)


def _dense0_kernel(x_ref, w_ref, as_ref, ad_ref, xs_ref, xd_ref):
    xp = jnp.dot(x_ref[...], w_ref[...], preferred_element_type=F32)
    xs_ref[:, : xp.shape[1]] = xp
    xs_ref[:, xp.shape[1]:] = jnp.dot(xp, as_ref[...], preferred_element_type=F32)
    xd_ref[...] = jnp.dot(xp, ad_ref[...], preferred_element_type=F32)


def _dense_mid_kernel(acc_ref, b_ref, w_ref, as_ref, adr_ref, xs_ref, xd_ref):
    HC = acc_ref.shape[1] // 2
    h = jnp.maximum(acc_ref[:, :HC] / (acc_ref[:, HC:] + EPS) + b_ref[...], 0.0)
    xp = jnp.dot(h, w_ref[...], preferred_element_type=F32)
    xs_ref[:, : xp.shape[1]] = xp
    xs_ref[:, xp.shape[1]:] = jnp.dot(xp, as_ref[...], preferred_element_type=F32)
    xd_ref[...] = jnp.dot(xp, adr_ref[...], preferred_element_type=F32)


def _make_edge_kernel(HC, CH, n_valid, lo, hi, Nh):
    def ek(src_ref, dst_ref, xs_ref, xd_ref, acc0_ref, acc1_ref):
        g = pl.program_id(0)

        @pl.when(g == 0)
        def _init():
            acc0_ref[...] = jnp.zeros_like(acc0_ref)
            acc1_ref[...] = jnp.zeros_like(acc1_ref)

        base = g * CH
        cmin = dst_ref[0, 0, 0]
        cmax = dst_ref[0, 0, CH - 1]

        @pl.when((cmax >= lo) & (cmin < hi))
        def _run():
            def one(j, acc_ref):
                s = src_ref[0, 0, j]
                d = dst_ref[0, 0, j]
                dl = jnp.clip(d - lo, 0, Nh - 1)
                rs = xs_ref[pl.ds(s, 1), :]
                rd = xd_ref[pl.ds(dl, 1), :]
                al = rs[:, HC:] + rd
                al = jnp.where(al > 0, al, 0.2 * al)
                ea = jnp.exp(al)
                valid = (d >= lo) & (d < hi) & (base + j < n_valid)
                ea = jnp.where(valid, ea, 0.0)
                upd = jnp.concatenate([ea * rs[:, :HC], ea], axis=1)
                acc_ref[pl.ds(dl, 1), :] = acc_ref[pl.ds(dl, 1), :] + upd

            def body(i, carry):
                one(2 * i, acc0_ref)
                one(2 * i + 1, acc1_ref)
                return carry

            jax.lax.fori_loop(0, CH // 2, body, 0)

    return ek


def _edge_pass(srcr, dstr, XS, XD, HC, CH, n_valid):
    N = XD.shape[0]
    G = srcr.shape[0]
    halves = []
    Nh = N // 2
    for lo in (0, Nh):
        ek = _make_edge_kernel(HC, CH, n_valid, lo, lo + Nh, Nh)
        acc_e, acc_o = pl.pallas_call(
            ek,
            grid=(G,),
            in_specs=[
                pl.BlockSpec((1, 1, CH), lambda g: (g, 0, 0), memory_space=pltpu.SMEM),
                pl.BlockSpec((1, 1, CH), lambda g: (g, 0, 0), memory_space=pltpu.SMEM),
                pl.BlockSpec((N, 2 * HC), lambda g: (0, 0)),
                pl.BlockSpec((Nh, HC), lambda g: (0, 0)),
            ],
            out_specs=[pl.BlockSpec((Nh, 2 * HC), lambda g: (0, 0)),
                       pl.BlockSpec((Nh, 2 * HC), lambda g: (0, 0))],
            out_shape=[jax.ShapeDtypeStruct((Nh, 2 * HC), F32),
                       jax.ShapeDtypeStruct((Nh, 2 * HC), F32)],
            compiler_params=pltpu.CompilerParams(
                vmem_limit_bytes=100 * 1024 * 1024,
            ),
        )(srcr, dstr, XS, jax.lax.slice_in_dim(XD, lo, lo + Nh))
        halves.append(acc_e + acc_o)
    return jnp.concatenate(halves, axis=0)


def _heads_kernel(acc_ref, b_ref, x_ref,
                  pw1_ref, pb1_ref, pw2_ref, pb2_ref, pw3_ref, pb3_ref,
                  cw1_ref, cb1_ref, cw2_ref, cb2_ref,
                  h_ref, pc_ref, pp_ref, mg_ref, cc_ref, op_ref):
    HC = acc_ref.shape[1] // 2
    h = acc_ref[:, :HC] / (acc_ref[:, HC:] + EPS) + b_ref[...]
    h_ref[...] = h
    p = jnp.maximum(jnp.dot(h, pw1_ref[...], preferred_element_type=F32) + pb1_ref[...], 0.0)
    p = jnp.maximum(jnp.dot(p, pw2_ref[...], preferred_element_type=F32) + pb2_ref[...], 0.0)
    p = jnp.dot(p, pw3_ref[...], preferred_element_type=F32) + pb3_ref[...]
    pc_ref[...] = p
    pp_ref[...] = p + x_ref[...]
    mags = jnp.sqrt(jnp.sum(p * p, axis=1, keepdims=True))
    mg8 = jnp.broadcast_to(mags, mg_ref.shape)
    mg_ref[...] = mg8
    c = jnp.maximum(jnp.dot(h, cw1_ref[...], preferred_element_type=F32) + cb1_ref[...], 0.0)
    c = jax.nn.sigmoid(jnp.dot(c, cw2_ref[...], preferred_element_type=F32) + cb2_ref[...])
    cc_ref[...] = c
    op_ref[...] = (mg8 >= 2.5).astype(jnp.int32)


def _full(shape):
    return pl.BlockSpec(shape, lambda i: tuple(0 for _ in shape))


@functools.partial(jax.jit, static_argnames=())
def kernel(x, edge_index, W0, as0, ad0, b0, W1, as1, ad1, b1, W2, as2, ad2, b2,
           pcW1, pcb1, pcW2, pcb2, pcW3, pcb3, ccW1, ccb1, ccW2, ccb2):
    N = x.shape[0]
    E = edge_index.shape[1]
    Bn = 2000 if N % 2000 == 0 else N
    CH = 1024
    n_valid = E + N

    ei = edge_index.astype(jnp.int32)
    loop = jnp.arange(N, dtype=jnp.int32)
    src = jnp.concatenate([ei[0], loop])
    dst = jnp.concatenate([ei[1], loop])
    order = jnp.argsort(dst)
    src = src[order]
    dst = dst[order]
    G = (n_valid + CH - 1) // CH
    pad = G * CH - n_valid
    srcr = jnp.pad(src, (0, pad)).reshape(G, 1, CH)
    dstr = jnp.pad(dst, (0, pad), constant_values=N - 1).reshape(G, 1, CH)

    xpad = jnp.pad(x.astype(F32), ((0, 0), (0, 1)))
    W0p = jnp.pad(W0.astype(F32), ((0, 1), (0, 0)))
    As0, Ad0 = _rep_mat(as0[0]), _rep_mat(ad0[0])
    As1, Ad1 = _rep_mat(as1[0]), _rep_mat(ad1[0])
    As2, Ad2 = _rep_mat(as2[0]), _rep_mat(ad2[0])
    b0r, b1r, b2r = b0.reshape(1, 64), b1.reshape(1, 64), b2.reshape(1, 32)

    grid_n = N // Bn

    XS0, XD0 = pl.pallas_call(
        _dense0_kernel,
        grid=(grid_n,),
        in_specs=[
            pl.BlockSpec((Bn, 8), lambda i: (i, 0)),
            _full((8, 64)), _full((64, 64)), _full((64, 64)),
        ],
        out_specs=[pl.BlockSpec((Bn, 128), lambda i: (i, 0)),
                   pl.BlockSpec((Bn, 64), lambda i: (i, 0))],
        out_shape=[jax.ShapeDtypeStruct((N, 128), F32),
                   jax.ShapeDtypeStruct((N, 64), F32)],
    )(xpad, W0p, As0, Ad0)

    acc0 = _edge_pass(srcr, dstr, XS0, XD0, 64, CH, n_valid)

    XS1, XD1 = pl.pallas_call(
        _dense_mid_kernel,
        grid=(grid_n,),
        in_specs=[
            pl.BlockSpec((Bn, 128), lambda i: (i, 0)),
            _full((1, 64)), _full((64, 64)), _full((64, 64)), _full((64, 64)),
        ],
        out_specs=[pl.BlockSpec((Bn, 128), lambda i: (i, 0)),
                   pl.BlockSpec((Bn, 64), lambda i: (i, 0))],
        out_shape=[jax.ShapeDtypeStruct((N, 128), F32),
                   jax.ShapeDtypeStruct((N, 64), F32)],
    )(acc0, b0r, W1.astype(F32), As1, Ad1)

    acc1 = _edge_pass(srcr, dstr, XS1, XD1, 64, CH, n_valid)

    XS2, XD2 = pl.pallas_call(
        _dense_mid_kernel,
        grid=(grid_n,),
        in_specs=[
            pl.BlockSpec((Bn, 128), lambda i: (i, 0)),
            _full((1, 64)), _full((64, 32)), _full((32, 32)), _full((32, 32)),
        ],
        out_specs=[pl.BlockSpec((Bn, 64), lambda i: (i, 0)),
                   pl.BlockSpec((Bn, 32), lambda i: (i, 0))],
        out_shape=[jax.ShapeDtypeStruct((N, 64), F32),
                   jax.ShapeDtypeStruct((N, 32), F32)],
    )(acc1, b1r, W2.astype(F32), As2, Ad2)

    acc2 = _edge_pass(srcr, dstr, XS2, XD2, 32, CH, n_valid)

    pcW3p = jnp.pad(pcW3.astype(F32), ((0, 0), (0, 5)))
    pcb3p = jnp.pad(pcb3.astype(F32), (0, 5)).reshape(1, 8)
    ccW2p = jnp.pad(ccW2.astype(F32), ((0, 0), (0, 7)))
    ccb2p = jnp.pad(ccb2.astype(F32), (0, 7)).reshape(1, 8)
    x3p = jnp.pad(x[:, :3].astype(F32), ((0, 0), (0, 5)))

    houts = pl.pallas_call(
        _heads_kernel,
        grid=(grid_n,),
        in_specs=[
            pl.BlockSpec((Bn, 64), lambda i: (i, 0)),
            _full((1, 32)),
            pl.BlockSpec((Bn, 8), lambda i: (i, 0)),
            _full((32, 32)), _full((1, 32)),
            _full((32, 16)), _full((1, 16)),
            _full((16, 8)), _full((1, 8)),
            _full((32, 8)), _full((1, 8)),
            _full((8, 8)), _full((1, 8)),
        ],
        out_specs=[pl.BlockSpec((Bn, w), lambda i: (i, 0)) for w in (32, 8, 8, 8, 8, 8)],
        out_shape=[
            jax.ShapeDtypeStruct((N, 32), F32),
            jax.ShapeDtypeStruct((N, 8), F32),
            jax.ShapeDtypeStruct((N, 8), F32),
            jax.ShapeDtypeStruct((N, 8), F32),
            jax.ShapeDtypeStruct((N, 8), F32),
            jax.ShapeDtypeStruct((N, 8), jnp.int32),
        ],
    )(acc2, b2r, x3p,
      pcW1.astype(F32), pcb1.reshape(1, 32), pcW2.astype(F32), pcb2.reshape(1, 16),
      pcW3p, pcb3p, ccW1.astype(F32), ccb1.reshape(1, 8), ccW2p, ccb2p)

    h_out, pc8, pp8, mg8, cc8, op8 = houts
    pc = pc8[:, :3]
    mags = mg8[:, 0]
    cc = cc8[:, :1]
    pred_pos = pp8[:, :3]
    node_ops = op8[:, 0]
    return (pc, mags, cc, pred_pos, node_ops, h_out)
